# baseline (speedup 1.0000x reference)
import functools

import jax
import jax.numpy as jnp
from jax import lax
from jax.experimental import pallas as pl
from jax.experimental.pallas import tpu as pltpu
from jax.experimental.pallas import tpu_sc as plsc

S = 2048
H = 768
F = 2048
NE = 8
TOPK = 2
A = S * TOPK
T = 1024
APAD = A + NE * T
GMAX = A // T + NE
EPAD = 128
SB = 128
NB = S // SB
SWL = 128
NC, NS, L = 2, 16, 16
NW = NC * NS


def _router_body(x_ref, wrt_ref, br_ref, w1_ref, w2_ref, p1_ref, p2_ref,
                 eid_ref, rblk_ref, gcnt_ref):
    logits = jnp.dot(x_ref[...], wrt_ref[...],
                     preferred_element_type=jnp.float32) + br_ref[...]
    lanes = lax.broadcasted_iota(jnp.int32, (S, EPAD), 1)
    m1 = jnp.max(logits, axis=1, keepdims=True)
    a1 = jnp.min(jnp.where(logits == m1, lanes, EPAD), axis=1, keepdims=True)
    l2 = jnp.where(lanes == a1, -jnp.inf, logits)
    m2 = jnp.max(l2, axis=1, keepdims=True)
    a2 = jnp.min(jnp.where(l2 == m2, lanes, EPAD), axis=1, keepdims=True)
    ed = jnp.exp(m2 - m1)
    w1 = 1.0 / (1.0 + ed)
    w1_ref[...] = jnp.broadcast_to(w1, (S, SWL))
    w2_ref[...] = jnp.broadcast_to(ed * w1, (S, SWL))

    oh1 = (lanes == a1).astype(jnp.float32)
    oh2 = (lanes == a2).astype(jnp.float32)
    tri = (lax.broadcasted_iota(jnp.int32, (SB, SB), 0)
           >= lax.broadcasted_iota(jnp.int32, (SB, SB), 1)).astype(jnp.float32)
    cum1_parts, cum2_parts = [], []
    run1 = jnp.zeros((1, EPAD), jnp.float32)
    run2 = jnp.zeros((1, EPAD), jnp.float32)
    for b in range(NB):
        blk1 = oh1[b * SB:(b + 1) * SB, :]
        blk2 = oh2[b * SB:(b + 1) * SB, :]
        c1b = jnp.dot(tri, blk1, preferred_element_type=jnp.float32)
        c2b = jnp.dot(tri, blk2, preferred_element_type=jnp.float32)
        cum1_parts.append(c1b + run1)
        cum2_parts.append(c2b + run2)
        run1 = run1 + c1b[SB - 1:SB, :]
        run2 = run2 + c2b[SB - 1:SB, :]
    cum1 = jnp.concatenate(cum1_parts, axis=0)
    cum2 = jnp.concatenate(cum2_parts, axis=0)
    counts = (run1 + run2).astype(jnp.int32)
    cnt1 = run1.astype(jnp.int32)

    rank1 = jnp.sum(jnp.where(lanes == a1, cum1, 0.0), axis=1,
                    keepdims=True).astype(jnp.int32) - 1
    cnt1_at_a2 = jnp.sum(jnp.where(lanes == a2,
                                   jnp.broadcast_to(cnt1.astype(jnp.float32),
                                                    (S, EPAD)), 0.0),
                         axis=1, keepdims=True).astype(jnp.int32)
    rank2 = cnt1_at_a2 + jnp.sum(jnp.where(lanes == a2, cum2, 0.0), axis=1,
                                 keepdims=True).astype(jnp.int32) - 1

    gs = lax.broadcasted_iota(jnp.int32, (1, EPAD), 1)
    zero_row = jnp.zeros((1, EPAD), jnp.int32)
    run_blk = zero_row[:, 0:1]
    run_rows = zero_row[:, 0:1]
    off1 = jnp.zeros((S, 1), jnp.int32)
    off2 = jnp.zeros((S, 1), jnp.int32)
    for e in range(NE):
        cnt_e = counts[0:1, e:e + 1]
        nblk_e = (cnt_e + (T - 1)) // T
        off1 = off1 + jnp.where(a1 == e, run_rows, 0)
        off2 = off2 + jnp.where(a2 == e, run_rows, 0)
        run_blk = run_blk + nblk_e
        run_rows = run_rows + nblk_e * T
    total_blk = run_blk
    gse = jnp.minimum(gs, total_blk - 1)
    eid = zero_row
    rblk = zero_row
    run_blk = zero_row[:, 0:1]
    run_rows = zero_row[:, 0:1]
    for e in range(NE):
        cnt_e = counts[0:1, e:e + 1]
        nblk_e = (cnt_e + (T - 1)) // T
        in_e = jnp.logical_and(gse >= run_blk, gse < run_blk + nblk_e)
        rblk = jnp.where(in_e, run_rows // T + (gse - run_blk), rblk)
        eid = jnp.where(in_e, e, eid)
        run_blk = run_blk + nblk_e
        run_rows = run_rows + nblk_e * T
    p1_ref[...] = off1 + rank1
    p2_ref[...] = off2 + rank2
    eid_ref[...] = eid
    rblk_ref[...] = rblk
    gcnt_ref[...] = jnp.broadcast_to(total_blk, (1, EPAD))


def _router(x2d, Wr, br):
    wrt = jnp.zeros((H, EPAD), jnp.float32).at[:, :NE].set(Wr.T)
    brp = jnp.full((1, EPAD), -1e30, jnp.float32).at[0, :NE].set(br)
    return pl.pallas_call(
        _router_body,
        out_shape=[
            jax.ShapeDtypeStruct((S, SWL), jnp.float32),
            jax.ShapeDtypeStruct((S, SWL), jnp.float32),
            jax.ShapeDtypeStruct((S, 1), jnp.int32),
            jax.ShapeDtypeStruct((S, 1), jnp.int32),
            jax.ShapeDtypeStruct((1, EPAD), jnp.int32),
            jax.ShapeDtypeStruct((1, EPAD), jnp.int32),
            jax.ShapeDtypeStruct((1, EPAD), jnp.int32),
        ],
    )(x2d, wrt, brp)


def _make_sc_dispatch():
    tpw = S // NW
    mesh = plsc.VectorSubcoreMesh(core_axis_name="c", subcore_axis_name="s")

    @functools.partial(
        pl.kernel, mesh=mesh,
        out_type=[
            jax.ShapeDtypeStruct((APAD, H), jnp.float32),
            jax.ShapeDtypeStruct((APAD, SWL), jnp.float32),
        ],
        scratch_types=[
            pltpu.VMEM((tpw,), jnp.int32),
            pltpu.VMEM((tpw,), jnp.int32),
            pltpu.VMEM((tpw, H), jnp.float32),
            pltpu.VMEM((tpw, SWL), jnp.float32),
            pltpu.VMEM((tpw, SWL), jnp.float32),
            pltpu.SemaphoreType.DMA,
            pltpu.SemaphoreType.DMA,
            pltpu.SemaphoreType.DMA,
            pltpu.SemaphoreType.DMA,
        ],
    )
    def dispatch_k(x_hbm, p1_hbm, p2_hbm, w1_hbm, w2_hbm, xs_hbm, sw_hbm,
                   p1v, p2v, rows, w1v, w2v, s1, s2, s3, s4):
        wid = lax.axis_index("s") * NC + lax.axis_index("c")
        base = wid * tpw
        pltpu.sync_copy(p1_hbm.at[pl.ds(base, tpw)], p1v)
        pltpu.sync_copy(p2_hbm.at[pl.ds(base, tpw)], p2v)
        pltpu.sync_copy(x_hbm.at[pl.ds(base, tpw)], rows)
        c1 = pltpu.async_copy(rows, xs_hbm.at[p1v], s1)
        c2 = pltpu.async_copy(rows, xs_hbm.at[p2v], s2)
        pltpu.sync_copy(w1_hbm.at[pl.ds(base, tpw)], w1v)
        pltpu.sync_copy(w2_hbm.at[pl.ds(base, tpw)], w2v)
        c3 = pltpu.async_copy(w1v, sw_hbm.at[p1v], s3)
        c4 = pltpu.async_copy(w2v, sw_hbm.at[p2v], s4)
        c1.wait()
        c2.wait()
        c3.wait()
        c4.wait()

    return dispatch_k


def _ffn_body(eid_ref, rblk_ref, gc_ref, xs_ref, sw_ref, w1_ref, w3_ref,
              w2_ref, ys_ref):
    g = pl.program_id(0)

    @pl.when(g < gc_ref[0])
    def _():
        xb = xs_ref[...]
        h1 = jnp.dot(xb, w1_ref[0], preferred_element_type=jnp.float32)
        h3 = jnp.dot(xb, w3_ref[0], preferred_element_type=jnp.float32)
        hid = h1 * jax.nn.sigmoid(h1) * h3
        yb = jnp.dot(hid, w2_ref[0], preferred_element_type=jnp.float32)
        ys_ref[...] = yb * sw_ref[:, 0:1]


def _ffn(eid, rblk, gcount, xs, swp, W1, W3, W2):
    grid_spec = pltpu.PrefetchScalarGridSpec(
        num_scalar_prefetch=3,
        grid=(GMAX,),
        in_specs=[
            pl.BlockSpec((T, H), lambda g, eid, rblk, gc: (rblk[g], 0)),
            pl.BlockSpec((T, SWL), lambda g, eid, rblk, gc: (rblk[g], 0)),
            pl.BlockSpec((1, H, F), lambda g, eid, rblk, gc: (eid[g], 0, 0)),
            pl.BlockSpec((1, H, F), lambda g, eid, rblk, gc: (eid[g], 0, 0)),
            pl.BlockSpec((1, F, H), lambda g, eid, rblk, gc: (eid[g], 0, 0)),
        ],
        out_specs=pl.BlockSpec((T, H), lambda g, eid, rblk, gc: (rblk[g], 0)),
    )
    return pl.pallas_call(
        _ffn_body,
        grid_spec=grid_spec,
        out_shape=jax.ShapeDtypeStruct((APAD, H), jnp.float32),
        compiler_params=pltpu.CompilerParams(
            dimension_semantics=("arbitrary",)),
    )(eid, rblk, gcount, xs, swp, W1, W3, W2)


def _make_sc_combine():
    tpw = S // NW
    mesh = plsc.VectorSubcoreMesh(core_axis_name="c", subcore_axis_name="s")

    @functools.partial(
        pl.kernel, mesh=mesh,
        out_type=jax.ShapeDtypeStruct((S, H), jnp.float32),
        scratch_types=[
            pltpu.VMEM((tpw,), jnp.int32),
            pltpu.VMEM((tpw,), jnp.int32),
            pltpu.VMEM((tpw, H), jnp.float32),
            pltpu.VMEM((tpw, H), jnp.float32),
            pltpu.SemaphoreType.DMA,
            pltpu.SemaphoreType.DMA,
        ],
    )
    def combine_k(ys_hbm, p1_hbm, p2_hbm, out_hbm, i1_v, i2_v, r1_v, r2_v,
                  sem1, sem2):
        wid = lax.axis_index("s") * NC + lax.axis_index("c")
        base = wid * tpw
        pltpu.sync_copy(p1_hbm.at[pl.ds(base, tpw)], i1_v)
        pltpu.sync_copy(p2_hbm.at[pl.ds(base, tpw)], i2_v)
        cp1 = pltpu.async_copy(ys_hbm.at[i1_v], r1_v, sem1)
        cp2 = pltpu.async_copy(ys_hbm.at[i2_v], r2_v, sem2)
        cp1.wait()
        cp2.wait()

        def _row(r, carry):
            for c in range(H // L):
                sl = pl.ds(c * L, L)
                r1_v[r, sl] = r1_v[r, sl] + r2_v[r, sl]
            return carry

        lax.fori_loop(0, tpw, _row, 0)
        pltpu.sync_copy(r1_v, out_hbm.at[pl.ds(base, tpw)])

    return combine_k


_sc_dispatch = _make_sc_dispatch()
_sc_combine = _make_sc_combine()


def kernel(x, Wr, br, W1, W3, W2):
    x2d = x.reshape(S, H)
    w1b, w2b, p1c, p2c, eidp, rblkp, gcntp = _router(x2d, Wr, br)
    p1 = p1c.reshape(S)
    p2 = p2c.reshape(S)
    xs, swp = _sc_dispatch(x2d, p1, p2, w1b, w2b)
    ys = _ffn(eidp.reshape(EPAD), rblkp.reshape(EPAD), gcntp.reshape(EPAD),
              xs, swp, W1, W3, W2)
    out = _sc_combine(ys, p1, p2)
    return out.reshape(x.shape)


# speedup vs baseline: 1.1454x; 1.1454x over previous
import functools

import jax
import jax.numpy as jnp
from jax import lax
from jax.experimental import pallas as pl
from jax.experimental.pallas import tpu as pltpu
from jax.experimental.pallas import tpu_sc as plsc

S = 2048
H = 768
F = 2048
NE = 8
TOPK = 2
A = S * TOPK
T = 768
GMAX = A // T + NE
APAD = GMAX * T
EPAD = 128
SB = 128
NB = S // SB
SWL = 128
NC, NS, L = 2, 16, 16
NW = NC * NS


def _router_body(x_ref, wrt_ref, br_ref, w1_ref, w2_ref, p1_ref, p2_ref,
                 eid_ref, rblk_ref, gcnt_ref):
    logits = jnp.dot(x_ref[...], wrt_ref[...],
                     preferred_element_type=jnp.float32) + br_ref[...]
    lanes = lax.broadcasted_iota(jnp.int32, (S, EPAD), 1)
    m1 = jnp.max(logits, axis=1, keepdims=True)
    a1 = jnp.min(jnp.where(logits == m1, lanes, EPAD), axis=1, keepdims=True)
    l2 = jnp.where(lanes == a1, -jnp.inf, logits)
    m2 = jnp.max(l2, axis=1, keepdims=True)
    a2 = jnp.min(jnp.where(l2 == m2, lanes, EPAD), axis=1, keepdims=True)
    ed = jnp.exp(m2 - m1)
    w1 = 1.0 / (1.0 + ed)
    w1_ref[...] = jnp.broadcast_to(w1, (S, SWL))
    w2_ref[...] = jnp.broadcast_to(ed * w1, (S, SWL))

    oh1 = (lanes == a1).astype(jnp.float32)
    oh2 = (lanes == a2).astype(jnp.float32)
    tri = (lax.broadcasted_iota(jnp.int32, (SB, SB), 0)
           >= lax.broadcasted_iota(jnp.int32, (SB, SB), 1)).astype(jnp.float32)
    cum1_parts, cum2_parts = [], []
    run1 = jnp.zeros((1, EPAD), jnp.float32)
    run2 = jnp.zeros((1, EPAD), jnp.float32)
    for b in range(NB):
        blk1 = oh1[b * SB:(b + 1) * SB, :]
        blk2 = oh2[b * SB:(b + 1) * SB, :]
        c1b = jnp.dot(tri, blk1, preferred_element_type=jnp.float32)
        c2b = jnp.dot(tri, blk2, preferred_element_type=jnp.float32)
        cum1_parts.append(c1b + run1)
        cum2_parts.append(c2b + run2)
        run1 = run1 + c1b[SB - 1:SB, :]
        run2 = run2 + c2b[SB - 1:SB, :]
    cum1 = jnp.concatenate(cum1_parts, axis=0)
    cum2 = jnp.concatenate(cum2_parts, axis=0)
    counts = (run1 + run2).astype(jnp.int32)
    cnt1 = run1.astype(jnp.int32)

    rank1 = jnp.sum(jnp.where(lanes == a1, cum1, 0.0), axis=1,
                    keepdims=True).astype(jnp.int32) - 1
    cnt1_at_a2 = jnp.sum(jnp.where(lanes == a2,
                                   jnp.broadcast_to(cnt1.astype(jnp.float32),
                                                    (S, EPAD)), 0.0),
                         axis=1, keepdims=True).astype(jnp.int32)
    rank2 = cnt1_at_a2 + jnp.sum(jnp.where(lanes == a2, cum2, 0.0), axis=1,
                                 keepdims=True).astype(jnp.int32) - 1

    gs = lax.broadcasted_iota(jnp.int32, (1, EPAD), 1)
    zero_row = jnp.zeros((1, EPAD), jnp.int32)
    run_blk = zero_row[:, 0:1]
    run_rows = zero_row[:, 0:1]
    off1 = jnp.zeros((S, 1), jnp.int32)
    off2 = jnp.zeros((S, 1), jnp.int32)
    for e in range(NE):
        cnt_e = counts[0:1, e:e + 1]
        nblk_e = (cnt_e + (T - 1)) // T
        off1 = off1 + jnp.where(a1 == e, run_rows, 0)
        off2 = off2 + jnp.where(a2 == e, run_rows, 0)
        run_blk = run_blk + nblk_e
        run_rows = run_rows + nblk_e * T
    total_blk = run_blk
    gse = jnp.minimum(gs, total_blk - 1)
    eid = zero_row
    rblk = zero_row
    run_blk = zero_row[:, 0:1]
    run_rows = zero_row[:, 0:1]
    for e in range(NE):
        cnt_e = counts[0:1, e:e + 1]
        nblk_e = (cnt_e + (T - 1)) // T
        in_e = jnp.logical_and(gse >= run_blk, gse < run_blk + nblk_e)
        rblk = jnp.where(in_e, run_rows // T + (gse - run_blk), rblk)
        eid = jnp.where(in_e, e, eid)
        run_blk = run_blk + nblk_e
        run_rows = run_rows + nblk_e * T
    p1_ref[...] = off1 + rank1
    p2_ref[...] = off2 + rank2
    eid_ref[...] = eid
    rblk_ref[...] = rblk
    gcnt_ref[...] = jnp.broadcast_to(total_blk, (1, EPAD))


def _router(x2d, Wr, br):
    wrt = jnp.zeros((H, EPAD), jnp.float32).at[:, :NE].set(Wr.T)
    brp = jnp.full((1, EPAD), -1e30, jnp.float32).at[0, :NE].set(br)
    return pl.pallas_call(
        _router_body,
        out_shape=[
            jax.ShapeDtypeStruct((S, SWL), jnp.float32),
            jax.ShapeDtypeStruct((S, SWL), jnp.float32),
            jax.ShapeDtypeStruct((S, 1), jnp.int32),
            jax.ShapeDtypeStruct((S, 1), jnp.int32),
            jax.ShapeDtypeStruct((1, EPAD), jnp.int32),
            jax.ShapeDtypeStruct((1, EPAD), jnp.int32),
            jax.ShapeDtypeStruct((1, EPAD), jnp.int32),
        ],
    )(x2d, wrt, brp)


def _make_sc_dispatch():
    tpw = S // NW
    mesh = plsc.VectorSubcoreMesh(core_axis_name="c", subcore_axis_name="s")

    @functools.partial(
        pl.kernel, mesh=mesh,
        out_type=[
            jax.ShapeDtypeStruct((APAD, H), jnp.float32),
            jax.ShapeDtypeStruct((APAD, SWL), jnp.float32),
        ],
        scratch_types=[
            pltpu.VMEM((tpw,), jnp.int32),
            pltpu.VMEM((tpw,), jnp.int32),
            pltpu.VMEM((tpw, H), jnp.float32),
            pltpu.VMEM((tpw, SWL), jnp.float32),
            pltpu.VMEM((tpw, SWL), jnp.float32),
            pltpu.SemaphoreType.DMA,
            pltpu.SemaphoreType.DMA,
            pltpu.SemaphoreType.DMA,
            pltpu.SemaphoreType.DMA,
        ],
    )
    def dispatch_k(x_hbm, p1_hbm, p2_hbm, w1_hbm, w2_hbm, xs_hbm, sw_hbm,
                   p1v, p2v, rows, w1v, w2v, s1, s2, s3, s4):
        wid = lax.axis_index("s") * NC + lax.axis_index("c")
        base = wid * tpw
        pltpu.sync_copy(p1_hbm.at[pl.ds(base, tpw)], p1v)
        pltpu.sync_copy(p2_hbm.at[pl.ds(base, tpw)], p2v)
        pltpu.sync_copy(x_hbm.at[pl.ds(base, tpw)], rows)
        c1 = pltpu.async_copy(rows, xs_hbm.at[p1v], s1)
        c2 = pltpu.async_copy(rows, xs_hbm.at[p2v], s2)
        pltpu.sync_copy(w1_hbm.at[pl.ds(base, tpw)], w1v)
        pltpu.sync_copy(w2_hbm.at[pl.ds(base, tpw)], w2v)
        c3 = pltpu.async_copy(w1v, sw_hbm.at[p1v], s3)
        c4 = pltpu.async_copy(w2v, sw_hbm.at[p2v], s4)
        c1.wait()
        c2.wait()
        c3.wait()
        c4.wait()

    return dispatch_k


def _ffn_body(eid_ref, rblk_ref, gc_ref, xs_ref, sw_ref, w1_ref, w3_ref,
              w2_ref, ys_ref):
    g = pl.program_id(0)

    @pl.when(g < gc_ref[0])
    def _():
        xb = xs_ref[...]
        h1 = jnp.dot(xb, w1_ref[0], preferred_element_type=jnp.float32)
        h3 = jnp.dot(xb, w3_ref[0], preferred_element_type=jnp.float32)
        hid = h1 * jax.nn.sigmoid(h1) * h3
        yb = jnp.dot(hid, w2_ref[0], preferred_element_type=jnp.float32)
        ys_ref[...] = yb * sw_ref[:, 0:1]


def _ffn(eid, rblk, gcount, xs, swp, W1, W3, W2):
    grid_spec = pltpu.PrefetchScalarGridSpec(
        num_scalar_prefetch=3,
        grid=(GMAX,),
        in_specs=[
            pl.BlockSpec((T, H), lambda g, eid, rblk, gc: (rblk[g], 0)),
            pl.BlockSpec((T, SWL), lambda g, eid, rblk, gc: (rblk[g], 0)),
            pl.BlockSpec((1, H, F), lambda g, eid, rblk, gc: (eid[g], 0, 0)),
            pl.BlockSpec((1, H, F), lambda g, eid, rblk, gc: (eid[g], 0, 0)),
            pl.BlockSpec((1, F, H), lambda g, eid, rblk, gc: (eid[g], 0, 0)),
        ],
        out_specs=pl.BlockSpec((T, H), lambda g, eid, rblk, gc: (rblk[g], 0)),
    )
    return pl.pallas_call(
        _ffn_body,
        grid_spec=grid_spec,
        out_shape=jax.ShapeDtypeStruct((APAD, H), jnp.float32),
        compiler_params=pltpu.CompilerParams(
            dimension_semantics=("arbitrary",)),
    )(eid, rblk, gcount, xs, swp, W1, W3, W2)


def _make_sc_combine():
    tpw = S // NW
    mesh = plsc.VectorSubcoreMesh(core_axis_name="c", subcore_axis_name="s")

    @functools.partial(
        pl.kernel, mesh=mesh,
        out_type=jax.ShapeDtypeStruct((S, H), jnp.float32),
        scratch_types=[
            pltpu.VMEM((tpw,), jnp.int32),
            pltpu.VMEM((tpw,), jnp.int32),
            pltpu.VMEM((tpw, H), jnp.float32),
            pltpu.VMEM((tpw, H), jnp.float32),
            pltpu.SemaphoreType.DMA,
            pltpu.SemaphoreType.DMA,
        ],
    )
    def combine_k(ys_hbm, p1_hbm, p2_hbm, out_hbm, i1_v, i2_v, r1_v, r2_v,
                  sem1, sem2):
        wid = lax.axis_index("s") * NC + lax.axis_index("c")
        base = wid * tpw
        pltpu.sync_copy(p1_hbm.at[pl.ds(base, tpw)], i1_v)
        pltpu.sync_copy(p2_hbm.at[pl.ds(base, tpw)], i2_v)
        cp1 = pltpu.async_copy(ys_hbm.at[i1_v], r1_v, sem1)
        cp2 = pltpu.async_copy(ys_hbm.at[i2_v], r2_v, sem2)
        cp1.wait()
        cp2.wait()

        def _row(r, carry):
            for c in range(H // L):
                sl = pl.ds(c * L, L)
                r1_v[r, sl] = r1_v[r, sl] + r2_v[r, sl]
            return carry

        lax.fori_loop(0, tpw, _row, 0)
        pltpu.sync_copy(r1_v, out_hbm.at[pl.ds(base, tpw)])

    return combine_k


_sc_dispatch = _make_sc_dispatch()
_sc_combine = _make_sc_combine()


def kernel(x, Wr, br, W1, W3, W2):
    x2d = x.reshape(S, H)
    w1b, w2b, p1c, p2c, eidp, rblkp, gcntp = _router(x2d, Wr, br)
    p1 = p1c.reshape(S)
    p2 = p2c.reshape(S)
    xs, swp = _sc_dispatch(x2d, p1, p2, w1b, w2b)
    ys = _ffn(eidp.reshape(EPAD), rblkp.reshape(EPAD), gcntp.reshape(EPAD),
              xs, swp, W1, W3, W2)
    out = _sc_combine(ys, p1, p2)
    return out.reshape(x.shape)


# speedup vs baseline: 1.2048x; 1.0519x over previous
import functools

import jax
import jax.numpy as jnp
from jax import lax
from jax.experimental import pallas as pl
from jax.experimental.pallas import tpu as pltpu
from jax.experimental.pallas import tpu_sc as plsc

S = 2048
H = 768
F = 2048
NE = 8
TOPK = 2
A = S * TOPK
T = 640
GMAX = A // T + NE
APAD = GMAX * T
EPAD = 128
SB = 128
NB = S // SB
SWL = 128
NC, NS, L = 2, 16, 16
NW = NC * NS


def _router_body(x_ref, wrt_ref, br_ref, w1_ref, w2_ref, p1_ref, p2_ref,
                 eid_ref, rblk_ref, gcnt_ref):
    logits = jnp.dot(x_ref[...], wrt_ref[...],
                     preferred_element_type=jnp.float32) + br_ref[...]
    lanes = lax.broadcasted_iota(jnp.int32, (S, EPAD), 1)
    m1 = jnp.max(logits, axis=1, keepdims=True)
    a1 = jnp.min(jnp.where(logits == m1, lanes, EPAD), axis=1, keepdims=True)
    l2 = jnp.where(lanes == a1, -jnp.inf, logits)
    m2 = jnp.max(l2, axis=1, keepdims=True)
    a2 = jnp.min(jnp.where(l2 == m2, lanes, EPAD), axis=1, keepdims=True)
    ed = jnp.exp(m2 - m1)
    w1 = 1.0 / (1.0 + ed)
    w1_ref[...] = jnp.broadcast_to(w1, (S, SWL))
    w2_ref[...] = jnp.broadcast_to(ed * w1, (S, SWL))

    oh1 = (lanes == a1).astype(jnp.float32)
    oh2 = (lanes == a2).astype(jnp.float32)
    tri = (lax.broadcasted_iota(jnp.int32, (SB, SB), 0)
           >= lax.broadcasted_iota(jnp.int32, (SB, SB), 1)).astype(jnp.float32)
    cum1_parts, cum2_parts = [], []
    run1 = jnp.zeros((1, EPAD), jnp.float32)
    run2 = jnp.zeros((1, EPAD), jnp.float32)
    for b in range(NB):
        blk1 = oh1[b * SB:(b + 1) * SB, :]
        blk2 = oh2[b * SB:(b + 1) * SB, :]
        c1b = jnp.dot(tri, blk1, preferred_element_type=jnp.float32)
        c2b = jnp.dot(tri, blk2, preferred_element_type=jnp.float32)
        cum1_parts.append(c1b + run1)
        cum2_parts.append(c2b + run2)
        run1 = run1 + c1b[SB - 1:SB, :]
        run2 = run2 + c2b[SB - 1:SB, :]
    cum1 = jnp.concatenate(cum1_parts, axis=0)
    cum2 = jnp.concatenate(cum2_parts, axis=0)
    counts = (run1 + run2).astype(jnp.int32)
    cnt1 = run1.astype(jnp.int32)

    rank1 = jnp.sum(jnp.where(lanes == a1, cum1, 0.0), axis=1,
                    keepdims=True).astype(jnp.int32) - 1
    cnt1_at_a2 = jnp.sum(jnp.where(lanes == a2,
                                   jnp.broadcast_to(cnt1.astype(jnp.float32),
                                                    (S, EPAD)), 0.0),
                         axis=1, keepdims=True).astype(jnp.int32)
    rank2 = cnt1_at_a2 + jnp.sum(jnp.where(lanes == a2, cum2, 0.0), axis=1,
                                 keepdims=True).astype(jnp.int32) - 1

    gs = lax.broadcasted_iota(jnp.int32, (1, EPAD), 1)
    zero_row = jnp.zeros((1, EPAD), jnp.int32)
    run_blk = zero_row[:, 0:1]
    run_rows = zero_row[:, 0:1]
    off1 = jnp.zeros((S, 1), jnp.int32)
    off2 = jnp.zeros((S, 1), jnp.int32)
    for e in range(NE):
        cnt_e = counts[0:1, e:e + 1]
        nblk_e = (cnt_e + (T - 1)) // T
        off1 = off1 + jnp.where(a1 == e, run_rows, 0)
        off2 = off2 + jnp.where(a2 == e, run_rows, 0)
        run_blk = run_blk + nblk_e
        run_rows = run_rows + nblk_e * T
    total_blk = run_blk
    gse = jnp.minimum(gs, total_blk - 1)
    eid = zero_row
    rblk = zero_row
    run_blk = zero_row[:, 0:1]
    run_rows = zero_row[:, 0:1]
    for e in range(NE):
        cnt_e = counts[0:1, e:e + 1]
        nblk_e = (cnt_e + (T - 1)) // T
        in_e = jnp.logical_and(gse >= run_blk, gse < run_blk + nblk_e)
        rblk = jnp.where(in_e, run_rows // T + (gse - run_blk), rblk)
        eid = jnp.where(in_e, e, eid)
        run_blk = run_blk + nblk_e
        run_rows = run_rows + nblk_e * T
    p1_ref[...] = off1 + rank1
    p2_ref[...] = off2 + rank2
    eid_ref[...] = eid
    rblk_ref[...] = rblk
    gcnt_ref[...] = jnp.broadcast_to(total_blk, (1, EPAD))


def _router(x2d, Wr, br):
    wrt = jnp.zeros((H, EPAD), jnp.float32).at[:, :NE].set(Wr.T)
    brp = jnp.full((1, EPAD), -1e30, jnp.float32).at[0, :NE].set(br)
    return pl.pallas_call(
        _router_body,
        out_shape=[
            jax.ShapeDtypeStruct((S, SWL), jnp.float32),
            jax.ShapeDtypeStruct((S, SWL), jnp.float32),
            jax.ShapeDtypeStruct((S, 1), jnp.int32),
            jax.ShapeDtypeStruct((S, 1), jnp.int32),
            jax.ShapeDtypeStruct((1, EPAD), jnp.int32),
            jax.ShapeDtypeStruct((1, EPAD), jnp.int32),
            jax.ShapeDtypeStruct((1, EPAD), jnp.int32),
        ],
    )(x2d, wrt, brp)


def _make_sc_dispatch():
    tpw = S // NW
    mesh = plsc.VectorSubcoreMesh(core_axis_name="c", subcore_axis_name="s")

    @functools.partial(
        pl.kernel, mesh=mesh,
        out_type=[
            jax.ShapeDtypeStruct((APAD, H), jnp.float32),
            jax.ShapeDtypeStruct((APAD, SWL), jnp.float32),
        ],
        scratch_types=[
            pltpu.VMEM((tpw,), jnp.int32),
            pltpu.VMEM((tpw,), jnp.int32),
            pltpu.VMEM((tpw, H), jnp.float32),
            pltpu.VMEM((tpw, SWL), jnp.float32),
            pltpu.VMEM((tpw, SWL), jnp.float32),
            pltpu.SemaphoreType.DMA,
            pltpu.SemaphoreType.DMA,
            pltpu.SemaphoreType.DMA,
            pltpu.SemaphoreType.DMA,
        ],
    )
    def dispatch_k(x_hbm, p1_hbm, p2_hbm, w1_hbm, w2_hbm, xs_hbm, sw_hbm,
                   p1v, p2v, rows, w1v, w2v, s1, s2, s3, s4):
        wid = lax.axis_index("s") * NC + lax.axis_index("c")
        base = wid * tpw
        pltpu.sync_copy(p1_hbm.at[pl.ds(base, tpw)], p1v)
        pltpu.sync_copy(p2_hbm.at[pl.ds(base, tpw)], p2v)
        pltpu.sync_copy(x_hbm.at[pl.ds(base, tpw)], rows)
        c1 = pltpu.async_copy(rows, xs_hbm.at[p1v], s1)
        c2 = pltpu.async_copy(rows, xs_hbm.at[p2v], s2)
        pltpu.sync_copy(w1_hbm.at[pl.ds(base, tpw)], w1v)
        pltpu.sync_copy(w2_hbm.at[pl.ds(base, tpw)], w2v)
        c3 = pltpu.async_copy(w1v, sw_hbm.at[p1v], s3)
        c4 = pltpu.async_copy(w2v, sw_hbm.at[p2v], s4)
        c1.wait()
        c2.wait()
        c3.wait()
        c4.wait()

    return dispatch_k


def _ffn_body(eid_ref, rblk_ref, gc_ref, xs_ref, sw_ref, w1_ref, w3_ref,
              w2_ref, ys_ref):
    g = pl.program_id(0)

    @pl.when(g < gc_ref[0])
    def _():
        xb = xs_ref[...]
        h1 = jnp.dot(xb, w1_ref[0], preferred_element_type=jnp.float32)
        h3 = jnp.dot(xb, w3_ref[0], preferred_element_type=jnp.float32)
        hid = h1 * jax.nn.sigmoid(h1) * h3
        yb = jnp.dot(hid, w2_ref[0], preferred_element_type=jnp.float32)
        ys_ref[...] = yb * sw_ref[:, 0:1]


def _ffn(eid, rblk, gcount, xs, swp, W1, W3, W2):
    grid_spec = pltpu.PrefetchScalarGridSpec(
        num_scalar_prefetch=3,
        grid=(GMAX,),
        in_specs=[
            pl.BlockSpec((T, H), lambda g, eid, rblk, gc: (rblk[g], 0)),
            pl.BlockSpec((T, SWL), lambda g, eid, rblk, gc: (rblk[g], 0)),
            pl.BlockSpec((1, H, F), lambda g, eid, rblk, gc: (eid[g], 0, 0)),
            pl.BlockSpec((1, H, F), lambda g, eid, rblk, gc: (eid[g], 0, 0)),
            pl.BlockSpec((1, F, H), lambda g, eid, rblk, gc: (eid[g], 0, 0)),
        ],
        out_specs=pl.BlockSpec((T, H), lambda g, eid, rblk, gc: (rblk[g], 0)),
    )
    return pl.pallas_call(
        _ffn_body,
        grid_spec=grid_spec,
        out_shape=jax.ShapeDtypeStruct((APAD, H), jnp.float32),
        compiler_params=pltpu.CompilerParams(
            dimension_semantics=("arbitrary",)),
    )(eid, rblk, gcount, xs, swp, W1, W3, W2)


def _make_sc_combine():
    tpw = S // NW
    mesh = plsc.VectorSubcoreMesh(core_axis_name="c", subcore_axis_name="s")

    @functools.partial(
        pl.kernel, mesh=mesh,
        out_type=jax.ShapeDtypeStruct((S, H), jnp.float32),
        scratch_types=[
            pltpu.VMEM((tpw,), jnp.int32),
            pltpu.VMEM((tpw,), jnp.int32),
            pltpu.VMEM((tpw, H), jnp.float32),
            pltpu.VMEM((tpw, H), jnp.float32),
            pltpu.SemaphoreType.DMA,
            pltpu.SemaphoreType.DMA,
        ],
    )
    def combine_k(ys_hbm, p1_hbm, p2_hbm, out_hbm, i1_v, i2_v, r1_v, r2_v,
                  sem1, sem2):
        wid = lax.axis_index("s") * NC + lax.axis_index("c")
        base = wid * tpw
        pltpu.sync_copy(p1_hbm.at[pl.ds(base, tpw)], i1_v)
        pltpu.sync_copy(p2_hbm.at[pl.ds(base, tpw)], i2_v)
        cp1 = pltpu.async_copy(ys_hbm.at[i1_v], r1_v, sem1)
        cp2 = pltpu.async_copy(ys_hbm.at[i2_v], r2_v, sem2)
        cp1.wait()
        cp2.wait()

        def _row(r, carry):
            for c in range(H // L):
                sl = pl.ds(c * L, L)
                r1_v[r, sl] = r1_v[r, sl] + r2_v[r, sl]
            return carry

        lax.fori_loop(0, tpw, _row, 0)
        pltpu.sync_copy(r1_v, out_hbm.at[pl.ds(base, tpw)])

    return combine_k


_sc_dispatch = _make_sc_dispatch()
_sc_combine = _make_sc_combine()


def kernel(x, Wr, br, W1, W3, W2):
    x2d = x.reshape(S, H)
    w1b, w2b, p1c, p2c, eidp, rblkp, gcntp = _router(x2d, Wr, br)
    p1 = p1c.reshape(S)
    p2 = p2c.reshape(S)
    xs, swp = _sc_dispatch(x2d, p1, p2, w1b, w2b)
    ys = _ffn(eidp.reshape(EPAD), rblkp.reshape(EPAD), gcntp.reshape(EPAD),
              xs, swp, W1, W3, W2)
    out = _sc_combine(ys, p1, p2)
    return out.reshape(x.shape)


# speedup vs baseline: 1.2357x; 1.0256x over previous
import functools

import jax
import jax.numpy as jnp
from jax import lax
from jax.experimental import pallas as pl
from jax.experimental.pallas import tpu as pltpu
from jax.experimental.pallas import tpu_sc as plsc

S = 2048
H = 768
F = 2048
NE = 8
TOPK = 2
A = S * TOPK
T = 576
GMAX = A // T + NE
APAD = GMAX * T
EPAD = 128
SB = 128
NB = S // SB
SWL = 128
NC, NS, L = 2, 16, 16
NW = NC * NS


def _router_body(x_ref, wrt_ref, br_ref, w1_ref, w2_ref, p1_ref, p2_ref,
                 eid_ref, rblk_ref, gcnt_ref):
    logits = jnp.dot(x_ref[...], wrt_ref[...],
                     preferred_element_type=jnp.float32) + br_ref[...]
    lanes = lax.broadcasted_iota(jnp.int32, (S, EPAD), 1)
    m1 = jnp.max(logits, axis=1, keepdims=True)
    a1 = jnp.min(jnp.where(logits == m1, lanes, EPAD), axis=1, keepdims=True)
    l2 = jnp.where(lanes == a1, -jnp.inf, logits)
    m2 = jnp.max(l2, axis=1, keepdims=True)
    a2 = jnp.min(jnp.where(l2 == m2, lanes, EPAD), axis=1, keepdims=True)
    ed = jnp.exp(m2 - m1)
    w1 = 1.0 / (1.0 + ed)
    w1_ref[...] = jnp.broadcast_to(w1, (S, SWL))
    w2_ref[...] = jnp.broadcast_to(ed * w1, (S, SWL))

    oh1 = (lanes == a1).astype(jnp.float32)
    oh2 = (lanes == a2).astype(jnp.float32)
    tri = (lax.broadcasted_iota(jnp.int32, (SB, SB), 0)
           >= lax.broadcasted_iota(jnp.int32, (SB, SB), 1)).astype(jnp.float32)
    cum1_parts, cum2_parts = [], []
    run1 = jnp.zeros((1, EPAD), jnp.float32)
    run2 = jnp.zeros((1, EPAD), jnp.float32)
    for b in range(NB):
        blk1 = oh1[b * SB:(b + 1) * SB, :]
        blk2 = oh2[b * SB:(b + 1) * SB, :]
        c1b = jnp.dot(tri, blk1, preferred_element_type=jnp.float32)
        c2b = jnp.dot(tri, blk2, preferred_element_type=jnp.float32)
        cum1_parts.append(c1b + run1)
        cum2_parts.append(c2b + run2)
        run1 = run1 + c1b[SB - 1:SB, :]
        run2 = run2 + c2b[SB - 1:SB, :]
    cum1 = jnp.concatenate(cum1_parts, axis=0)
    cum2 = jnp.concatenate(cum2_parts, axis=0)
    counts = (run1 + run2).astype(jnp.int32)
    cnt1 = run1.astype(jnp.int32)

    rank1 = jnp.sum(jnp.where(lanes == a1, cum1, 0.0), axis=1,
                    keepdims=True).astype(jnp.int32) - 1
    cnt1_at_a2 = jnp.sum(jnp.where(lanes == a2,
                                   jnp.broadcast_to(cnt1.astype(jnp.float32),
                                                    (S, EPAD)), 0.0),
                         axis=1, keepdims=True).astype(jnp.int32)
    rank2 = cnt1_at_a2 + jnp.sum(jnp.where(lanes == a2, cum2, 0.0), axis=1,
                                 keepdims=True).astype(jnp.int32) - 1

    gs = lax.broadcasted_iota(jnp.int32, (1, EPAD), 1)
    zero_row = jnp.zeros((1, EPAD), jnp.int32)
    run_blk = zero_row[:, 0:1]
    run_rows = zero_row[:, 0:1]
    off1 = jnp.zeros((S, 1), jnp.int32)
    off2 = jnp.zeros((S, 1), jnp.int32)
    for e in range(NE):
        cnt_e = counts[0:1, e:e + 1]
        nblk_e = (cnt_e + (T - 1)) // T
        off1 = off1 + jnp.where(a1 == e, run_rows, 0)
        off2 = off2 + jnp.where(a2 == e, run_rows, 0)
        run_blk = run_blk + nblk_e
        run_rows = run_rows + nblk_e * T
    total_blk = run_blk
    gse = jnp.minimum(gs, total_blk - 1)
    eid = zero_row
    rblk = zero_row
    run_blk = zero_row[:, 0:1]
    run_rows = zero_row[:, 0:1]
    for e in range(NE):
        cnt_e = counts[0:1, e:e + 1]
        nblk_e = (cnt_e + (T - 1)) // T
        in_e = jnp.logical_and(gse >= run_blk, gse < run_blk + nblk_e)
        rblk = jnp.where(in_e, run_rows // T + (gse - run_blk), rblk)
        eid = jnp.where(in_e, e, eid)
        run_blk = run_blk + nblk_e
        run_rows = run_rows + nblk_e * T
    p1_ref[...] = off1 + rank1
    p2_ref[...] = off2 + rank2
    eid_ref[...] = eid
    rblk_ref[...] = rblk
    gcnt_ref[...] = jnp.broadcast_to(total_blk, (1, EPAD))


def _router(x2d, Wr, br):
    wrt = jnp.zeros((H, EPAD), jnp.float32).at[:, :NE].set(Wr.T)
    brp = jnp.full((1, EPAD), -1e30, jnp.float32).at[0, :NE].set(br)
    return pl.pallas_call(
        _router_body,
        out_shape=[
            jax.ShapeDtypeStruct((S, SWL), jnp.float32),
            jax.ShapeDtypeStruct((S, SWL), jnp.float32),
            jax.ShapeDtypeStruct((S, 1), jnp.int32),
            jax.ShapeDtypeStruct((S, 1), jnp.int32),
            jax.ShapeDtypeStruct((1, EPAD), jnp.int32),
            jax.ShapeDtypeStruct((1, EPAD), jnp.int32),
            jax.ShapeDtypeStruct((1, EPAD), jnp.int32),
        ],
    )(x2d, wrt, brp)


def _make_sc_dispatch():
    tpw = S // NW
    mesh = plsc.VectorSubcoreMesh(core_axis_name="c", subcore_axis_name="s")

    @functools.partial(
        pl.kernel, mesh=mesh,
        out_type=[
            jax.ShapeDtypeStruct((APAD, H), jnp.float32),
            jax.ShapeDtypeStruct((APAD, SWL), jnp.float32),
        ],
        scratch_types=[
            pltpu.VMEM((tpw,), jnp.int32),
            pltpu.VMEM((tpw,), jnp.int32),
            pltpu.VMEM((tpw, H), jnp.float32),
            pltpu.VMEM((tpw, SWL), jnp.float32),
            pltpu.VMEM((tpw, SWL), jnp.float32),
            pltpu.SemaphoreType.DMA,
            pltpu.SemaphoreType.DMA,
            pltpu.SemaphoreType.DMA,
            pltpu.SemaphoreType.DMA,
        ],
    )
    def dispatch_k(x_hbm, p1_hbm, p2_hbm, w1_hbm, w2_hbm, xs_hbm, sw_hbm,
                   p1v, p2v, rows, w1v, w2v, s1, s2, s3, s4):
        wid = lax.axis_index("s") * NC + lax.axis_index("c")
        base = wid * tpw
        pltpu.sync_copy(p1_hbm.at[pl.ds(base, tpw)], p1v)
        pltpu.sync_copy(p2_hbm.at[pl.ds(base, tpw)], p2v)
        pltpu.sync_copy(x_hbm.at[pl.ds(base, tpw)], rows)
        c1 = pltpu.async_copy(rows, xs_hbm.at[p1v], s1)
        c2 = pltpu.async_copy(rows, xs_hbm.at[p2v], s2)
        pltpu.sync_copy(w1_hbm.at[pl.ds(base, tpw)], w1v)
        pltpu.sync_copy(w2_hbm.at[pl.ds(base, tpw)], w2v)
        c3 = pltpu.async_copy(w1v, sw_hbm.at[p1v], s3)
        c4 = pltpu.async_copy(w2v, sw_hbm.at[p2v], s4)
        c1.wait()
        c2.wait()
        c3.wait()
        c4.wait()

    return dispatch_k


def _ffn_body(eid_ref, rblk_ref, gc_ref, xs_ref, sw_ref, w1_ref, w3_ref,
              w2_ref, ys_ref):
    g = pl.program_id(0)

    @pl.when(g < gc_ref[0])
    def _():
        xb = xs_ref[...]
        h1 = jnp.dot(xb, w1_ref[0], preferred_element_type=jnp.float32)
        h3 = jnp.dot(xb, w3_ref[0], preferred_element_type=jnp.float32)
        hid = h1 * jax.nn.sigmoid(h1) * h3
        yb = jnp.dot(hid, w2_ref[0], preferred_element_type=jnp.float32)
        ys_ref[...] = yb * sw_ref[:, 0:1]


def _ffn(eid, rblk, gcount, xs, swp, W1, W3, W2):
    grid_spec = pltpu.PrefetchScalarGridSpec(
        num_scalar_prefetch=3,
        grid=(GMAX,),
        in_specs=[
            pl.BlockSpec((T, H), lambda g, eid, rblk, gc: (rblk[g], 0)),
            pl.BlockSpec((T, SWL), lambda g, eid, rblk, gc: (rblk[g], 0)),
            pl.BlockSpec((1, H, F), lambda g, eid, rblk, gc: (eid[g], 0, 0)),
            pl.BlockSpec((1, H, F), lambda g, eid, rblk, gc: (eid[g], 0, 0)),
            pl.BlockSpec((1, F, H), lambda g, eid, rblk, gc: (eid[g], 0, 0)),
        ],
        out_specs=pl.BlockSpec((T, H), lambda g, eid, rblk, gc: (rblk[g], 0)),
    )
    return pl.pallas_call(
        _ffn_body,
        grid_spec=grid_spec,
        out_shape=jax.ShapeDtypeStruct((APAD, H), jnp.float32),
        compiler_params=pltpu.CompilerParams(
            dimension_semantics=("arbitrary",)),
    )(eid, rblk, gcount, xs, swp, W1, W3, W2)


def _make_sc_combine():
    tpw = S // NW
    mesh = plsc.VectorSubcoreMesh(core_axis_name="c", subcore_axis_name="s")

    @functools.partial(
        pl.kernel, mesh=mesh,
        out_type=jax.ShapeDtypeStruct((S, H), jnp.float32),
        scratch_types=[
            pltpu.VMEM((tpw,), jnp.int32),
            pltpu.VMEM((tpw,), jnp.int32),
            pltpu.VMEM((tpw, H), jnp.float32),
            pltpu.VMEM((tpw, H), jnp.float32),
            pltpu.SemaphoreType.DMA,
            pltpu.SemaphoreType.DMA,
        ],
    )
    def combine_k(ys_hbm, p1_hbm, p2_hbm, out_hbm, i1_v, i2_v, r1_v, r2_v,
                  sem1, sem2):
        wid = lax.axis_index("s") * NC + lax.axis_index("c")
        base = wid * tpw
        pltpu.sync_copy(p1_hbm.at[pl.ds(base, tpw)], i1_v)
        pltpu.sync_copy(p2_hbm.at[pl.ds(base, tpw)], i2_v)
        cp1 = pltpu.async_copy(ys_hbm.at[i1_v], r1_v, sem1)
        cp2 = pltpu.async_copy(ys_hbm.at[i2_v], r2_v, sem2)
        cp1.wait()
        cp2.wait()

        def _row(r, carry):
            for c in range(H // L):
                sl = pl.ds(c * L, L)
                r1_v[r, sl] = r1_v[r, sl] + r2_v[r, sl]
            return carry

        lax.fori_loop(0, tpw, _row, 0)
        pltpu.sync_copy(r1_v, out_hbm.at[pl.ds(base, tpw)])

    return combine_k


_sc_dispatch = _make_sc_dispatch()
_sc_combine = _make_sc_combine()


def kernel(x, Wr, br, W1, W3, W2):
    x2d = x.reshape(S, H)
    w1b, w2b, p1c, p2c, eidp, rblkp, gcntp = _router(x2d, Wr, br)
    p1 = p1c.reshape(S)
    p2 = p2c.reshape(S)
    xs, swp = _sc_dispatch(x2d, p1, p2, w1b, w2b)
    ys = _ffn(eidp.reshape(EPAD), rblkp.reshape(EPAD), gcntp.reshape(EPAD),
              xs, swp, W1, W3, W2)
    out = _sc_combine(ys, p1, p2)
    return out.reshape(x.shape)
